# baseline (speedup 1.0000x reference)
import functools

import jax
import jax.numpy as jnp
from jax import lax
from jax.experimental import pallas as pl
from jax.experimental.pallas import tpu as pltpu
from jax.experimental.pallas import tpu_sc as plsc

_E = 64
_TB = 1024
_NC = 2
_NS = 16
_NW = _NC * _NS
_L = 16


def _matmul_body(x_ref, w_ref, lg_ref):
    lg_ref[...] = jax.lax.dot_general(
        w_ref[...], x_ref[...], (((1,), (1,)), ((), ())),
        preferred_element_type=jnp.float32)


def _gate_logits_t(x, W, chunk, nchunks):
    n, d = x.shape
    csz = n // nchunks
    steps = csz // _TB
    return pl.pallas_call(
        _matmul_body,
        grid=(steps,),
        in_specs=[
            pl.BlockSpec((_TB, d), lambda i: (i + chunk * steps, 0)),
            pl.BlockSpec((_E, d), lambda i: (0, 0)),
        ],
        out_specs=pl.BlockSpec((_E, _TB), lambda i: (0, i)),
        out_shape=jax.ShapeDtypeStruct((_E, csz), jnp.float32),
        compiler_params=pltpu.CompilerParams(
            dimension_semantics=("arbitrary",),
        ),
    )(x, W)


def _sc_top2_body(tpw, lg_hbm, out_hbm, lg_v, o_v):
    wid = lax.axis_index("s") * _NC + lax.axis_index("c")
    base = wid * tpw
    pltpu.sync_copy(lg_hbm.at[:, pl.ds(base, tpw)], lg_v)

    def group(g):
        tok = g * _L
        neg = jnp.full((_L,), -jnp.inf, jnp.float32)
        zero = jnp.zeros((_L,), jnp.int32)
        bestv, secondv = neg, neg
        besti, secondi = zero, zero
        for e in range(_E):
            col = jnp.full((_L,), e, jnp.int32)
            v = lg_v[e, pl.ds(tok, _L)]
            gt1 = v > bestv
            gt2 = v > secondv
            sv = jnp.maximum(secondv, v)
            si = jnp.where(gt2, col, secondi)
            secondv = jnp.where(gt1, bestv, sv)
            secondi = jnp.where(gt1, besti, si)
            bestv = jnp.maximum(bestv, v)
            besti = jnp.where(gt1, col, besti)
        ex = jnp.exp(secondv - bestv)
        w0 = 1.0 / (1.0 + ex)
        o_v[0, pl.ds(tok, _L)] = besti.astype(jnp.float32)
        o_v[1, pl.ds(tok, _L)] = secondi.astype(jnp.float32)
        o_v[2, pl.ds(tok, _L)] = w0
        o_v[3, pl.ds(tok, _L)] = 1.0 - w0

    plsc.parallel_loop(0, tpw // _L, unroll=8)(group)
    pltpu.sync_copy(o_v, out_hbm.at[:, pl.ds(base, tpw)])


def _sc_top2(logits_t):
    n = logits_t.shape[1]
    tpw = n // _NW
    mesh = plsc.VectorSubcoreMesh(core_axis_name="c", subcore_axis_name="s")
    fn = pl.kernel(
        functools.partial(_sc_top2_body, tpw),
        out_type=jax.ShapeDtypeStruct((4, n), jnp.float32),
        mesh=mesh,
        scratch_types=[
            pltpu.VMEM((_E, tpw), jnp.float32),
            pltpu.VMEM((4, tpw), jnp.float32),
        ],
    )
    return fn(logits_t)


_NCHUNK = 1


def kernel(x, W):
    parts = [_sc_top2(_gate_logits_t(x, W, c, _NCHUNK)) for c in range(_NCHUNK)]
    out = jnp.concatenate(parts, axis=1)
    topi = out[0:2].T.astype(jnp.int32)
    weights = out[2:4].T
    return (topi, weights)


# speedup vs baseline: 1.0226x; 1.0226x over previous
import functools

import jax
import jax.numpy as jnp
from jax import lax
from jax.experimental import pallas as pl
from jax.experimental.pallas import tpu as pltpu
from jax.experimental.pallas import tpu_sc as plsc

_E = 64
_TB = 1024
_NC = 2
_NS = 16
_NW = _NC * _NS
_L = 16


def _matmul_body(x_ref, w_ref, lg_ref):
    lg_ref[...] = jax.lax.dot_general(
        w_ref[...], x_ref[...], (((1,), (1,)), ((), ())),
        preferred_element_type=jnp.float32)


def _gate_logits_t(x, W, chunk, nchunks):
    n, d = x.shape
    csz = n // nchunks
    steps = csz // _TB
    return pl.pallas_call(
        _matmul_body,
        grid=(steps,),
        in_specs=[
            pl.BlockSpec((_TB, d), lambda i: (i + chunk * steps, 0)),
            pl.BlockSpec((_E, d), lambda i: (0, 0)),
        ],
        out_specs=pl.BlockSpec((_E, _TB), lambda i: (0, i)),
        out_shape=jax.ShapeDtypeStruct((_E, csz), jnp.float32),
        compiler_params=pltpu.CompilerParams(
            dimension_semantics=("arbitrary",),
        ),
    )(x, W)


def _sc_top2_body(tpw, lg_hbm, out_hbm, lg_v, o_v):
    wid = lax.axis_index("s") * _NC + lax.axis_index("c")
    base = wid * tpw
    pltpu.sync_copy(lg_hbm.at[:, pl.ds(base, tpw)], lg_v)

    def group(g):
        tok = g * _L
        neg = jnp.full((_L,), -jnp.inf, jnp.float32)
        zero = jnp.zeros((_L,), jnp.int32)
        bestv, secondv = neg, neg
        besti, secondi = zero, zero
        for e in range(_E):
            col = jnp.full((_L,), e, jnp.int32)
            v = lg_v[e, pl.ds(tok, _L)]
            gt1 = v > bestv
            gt2 = v > secondv
            sv = jnp.maximum(secondv, v)
            si = jnp.where(gt2, col, secondi)
            secondv = jnp.where(gt1, bestv, sv)
            secondi = jnp.where(gt1, besti, si)
            bestv = jnp.maximum(bestv, v)
            besti = jnp.where(gt1, col, besti)
        ex = jnp.exp(secondv - bestv)
        w0 = 1.0 / (1.0 + ex)
        o_v[0, pl.ds(tok, _L)] = besti.astype(jnp.float32)
        o_v[1, pl.ds(tok, _L)] = secondi.astype(jnp.float32)
        o_v[2, pl.ds(tok, _L)] = w0
        o_v[3, pl.ds(tok, _L)] = 1.0 - w0

    plsc.parallel_loop(0, tpw // _L, unroll=4)(group)
    pltpu.sync_copy(o_v, out_hbm.at[:, pl.ds(base, tpw)])


def _sc_top2(logits_t):
    n = logits_t.shape[1]
    tpw = n // _NW
    mesh = plsc.VectorSubcoreMesh(core_axis_name="c", subcore_axis_name="s")
    fn = pl.kernel(
        functools.partial(_sc_top2_body, tpw),
        out_type=jax.ShapeDtypeStruct((4, n), jnp.float32),
        mesh=mesh,
        scratch_types=[
            pltpu.VMEM((_E, tpw), jnp.float32),
            pltpu.VMEM((4, tpw), jnp.float32),
        ],
    )
    return fn(logits_t)


_NCHUNK = 1


def kernel(x, W):
    parts = [_sc_top2(_gate_logits_t(x, W, c, _NCHUNK)) for c in range(_NCHUNK)]
    out = jnp.concatenate(parts, axis=1)
    topi = out[0:2].T.astype(jnp.int32)
    weights = out[2:4].T
    return (topi, weights)


# speedup vs baseline: 1.0278x; 1.0051x over previous
import functools

import jax
import jax.numpy as jnp
from jax import lax
from jax.experimental import pallas as pl
from jax.experimental.pallas import tpu as pltpu
from jax.experimental.pallas import tpu_sc as plsc

_E = 64
_TB = 1024
_NC = 2
_NS = 16
_NW = _NC * _NS
_L = 16


def _matmul_body(tpw, x_ref, w_ref, lg_ref):
    lg = jax.lax.dot_general(
        w_ref[...], x_ref[...], (((1,), (1,)), ((), ())),
        preferred_element_type=jnp.float32)
    for s in range(_TB // tpw):
        lg_ref[s] = lg[:, s * tpw:(s + 1) * tpw]


def _gate_logits_t(x, W):
    n, d = x.shape
    tpw = n // _NW
    slabs_per_step = _TB // tpw
    return pl.pallas_call(
        functools.partial(_matmul_body, tpw),
        grid=(n // _TB,),
        in_specs=[
            pl.BlockSpec((_TB, d), lambda i: (i, 0)),
            pl.BlockSpec((_E, d), lambda i: (0, 0)),
        ],
        out_specs=pl.BlockSpec((slabs_per_step, _E, tpw), lambda i: (i, 0, 0)),
        out_shape=jax.ShapeDtypeStruct((_NW, _E, tpw), jnp.float32),
        compiler_params=pltpu.CompilerParams(
            dimension_semantics=("arbitrary",),
        ),
    )(x, W)


def _sc_top2_body(tpw, lg_hbm, out_hbm, lg_v, o_v):
    wid = lax.axis_index("s") * _NC + lax.axis_index("c")
    base = wid * tpw
    pltpu.sync_copy(lg_hbm.at[wid], lg_v)

    def group(g):
        tok = g * _L
        neg = jnp.full((_L,), -jnp.inf, jnp.float32)
        zero = jnp.zeros((_L,), jnp.int32)
        bestv, secondv = neg, neg
        besti, secondi = zero, zero
        for e in range(_E):
            col = jnp.full((_L,), e, jnp.int32)
            v = lg_v[e, pl.ds(tok, _L)]
            gt1 = v > bestv
            gt2 = v > secondv
            sv = jnp.maximum(secondv, v)
            si = jnp.where(gt2, col, secondi)
            secondv = jnp.where(gt1, bestv, sv)
            secondi = jnp.where(gt1, besti, si)
            bestv = jnp.maximum(bestv, v)
            besti = jnp.where(gt1, col, besti)
        ex = jnp.exp(secondv - bestv)
        w0 = 1.0 / (1.0 + ex)
        o_v[0, pl.ds(tok, _L)] = besti.astype(jnp.float32)
        o_v[1, pl.ds(tok, _L)] = secondi.astype(jnp.float32)
        o_v[2, pl.ds(tok, _L)] = w0
        o_v[3, pl.ds(tok, _L)] = 1.0 - w0

    plsc.parallel_loop(0, tpw // _L, unroll=4)(group)
    pltpu.sync_copy(o_v, out_hbm.at[:, pl.ds(base, tpw)])


def _sc_top2(logits_b):
    nw, e, tpw = logits_b.shape
    n = nw * tpw
    mesh = plsc.VectorSubcoreMesh(core_axis_name="c", subcore_axis_name="s")
    fn = pl.kernel(
        functools.partial(_sc_top2_body, tpw),
        out_type=jax.ShapeDtypeStruct((4, n), jnp.float32),
        mesh=mesh,
        scratch_types=[
            pltpu.VMEM((_E, tpw), jnp.float32),
            pltpu.VMEM((4, tpw), jnp.float32),
        ],
    )
    return fn(logits_b)


def kernel(x, W):
    out = _sc_top2(_gate_logits_t(x, W))
    topi = out[0:2].T.astype(jnp.int32)
    weights = out[2:4].T
    return (topi, weights)
